# baseline (speedup 1.0000x reference)
import functools

import jax
import jax.numpy as jnp
from jax.experimental import pallas as pl
from jax.experimental.pallas import tpu as pltpu

_BBLK = 32
_NBUF = 4


def _body(nbb, nsteps, otu_ref, clr_ref, sel_ref, w_ref, b_ref, out_ref,
          tpb_ref):
    i = pl.program_id(0)
    bblk = out_ref.shape[0]
    g_idx = jax.lax.rem(i, nbb)

    @pl.when(g_idx == 0)
    def _fold_bias():
        tpb_ref[...] = otu_ref[...] + b_ref[...]

    blk = clr_ref[0]
    if bblk == blk.shape[1]:
        cols = blk
    else:
        cols = jnp.dot(
            blk, sel_ref[0],
            preferred_element_type=jnp.float32,
            precision=jax.lax.Precision.HIGHEST,
        )
    for j in range(bblk):
        out_ref[j] = tpb_ref[...] + cols[:, j:j + 1] * w_ref[...]


def _pick_dblk(d: int) -> int:
    best = 8
    for cand in range(8, 513, 8):
        if d % cand == 0:
            best = cand
    return best


def kernel(clr, otu_table, W_val, b_val):
    B, D = clr.shape
    E = otu_table.shape[1]
    dblk = _pick_dblk(D)
    ndb = D // dblk
    bblk = _BBLK if B % _BBLK == 0 else 1
    nbb = B // bblk
    nsteps = ndb * nbb

    clr3 = clr.T.reshape(ndb, dblk, B)
    w_row = W_val[:, 0].reshape(1, E)
    b_row = b_val.reshape(1, E)
    sel3 = (
        jax.lax.broadcasted_iota(jnp.int32, (nbb, B, bblk), 1)
        == jax.lax.broadcasted_iota(jnp.int32, (nbb, B, bblk), 2)
        + jax.lax.broadcasted_iota(jnp.int32, (nbb, B, bblk), 0) * bblk
    ).astype(jnp.float32)

    out = pl.pallas_call(
        functools.partial(_body, nbb, nsteps),
        grid=(nsteps,),
        in_specs=[
            pl.BlockSpec((dblk, E), lambda i: (i // nbb, 0)),
            pl.BlockSpec((1, dblk, B), lambda i: (i // nbb, 0, 0)),
            pl.BlockSpec((1, B, bblk), lambda i: (i % nbb, 0, 0)),
            pl.BlockSpec((1, E), lambda i: (0, 0)),
            pl.BlockSpec((1, E), lambda i: (0, 0)),
        ],
        out_specs=pl.BlockSpec(
            (bblk, dblk, E), lambda i: (i % nbb, i // nbb, 0)),
        out_shape=jax.ShapeDtypeStruct((B, D, E), jnp.float32),
        scratch_shapes=[
            pltpu.VMEM((dblk, E), jnp.float32),
        ],
    )(otu_table, clr3, sel3, w_row, b_row)
    return out


# speedup vs baseline: 1.0146x; 1.0146x over previous
import functools

import jax
import jax.numpy as jnp
from jax.experimental import pallas as pl
from jax.experimental.pallas import tpu as pltpu

_BBLK = 32
_NBUF = 2


def _body(nbb, nsteps, otu_ref, clr_ref, sel_ref, w_ref, b_ref, out_ref,
          buf_ref, tpb_ref, sems):
    i = pl.program_id(0)
    bblk = buf_ref.shape[1]
    dblk = buf_ref.shape[2]
    slot = jax.lax.rem(i, _NBUF)
    g_idx = jax.lax.rem(i, nbb)

    dst0 = out_ref.at[0, pl.ds(0, dblk), :]

    @pl.when(i >= _NBUF)
    def _wait_prev():
        for j in range(bblk):
            pltpu.make_async_copy(buf_ref.at[slot, j], dst0, sems.at[slot]).wait()

    @pl.when(g_idx == 0)
    def _fold_bias():
        tpb_ref[...] = otu_ref[...] + b_ref[...]

    blk = clr_ref[0]
    if bblk == blk.shape[1]:
        cols = blk
    else:
        cols = jnp.dot(
            blk, sel_ref[0],
            preferred_element_type=jnp.float32,
            precision=jax.lax.Precision.HIGHEST,
        )
    for j in range(bblk):
        buf_ref[slot, j] = tpb_ref[...] + cols[:, j:j + 1] * w_ref[...]

    d_idx = i // nbb
    for j in range(bblk):
        dst = out_ref.at[g_idx * bblk + j, pl.ds(d_idx * dblk, dblk), :]
        pltpu.make_async_copy(buf_ref.at[slot, j], dst, sems.at[slot]).start()

    @pl.when(i == nsteps - 1)
    def _drain():
        for k in range(_NBUF):
            for j in range(bblk):
                pltpu.make_async_copy(buf_ref.at[k, j], dst0, sems.at[k]).wait()


def _pick_dblk(d: int) -> int:
    best = 8
    for cand in range(8, 513, 8):
        if d % cand == 0:
            best = cand
    return best


def kernel(clr, otu_table, W_val, b_val):
    B, D = clr.shape
    E = otu_table.shape[1]
    dblk = _pick_dblk(D)
    ndb = D // dblk
    bblk = _BBLK if B % _BBLK == 0 else 1
    nbb = B // bblk
    nsteps = ndb * nbb

    clr3 = clr.T.reshape(ndb, dblk, B)
    w_row = W_val[:, 0].reshape(1, E)
    b_row = b_val.reshape(1, E)
    sel3 = (
        jax.lax.broadcasted_iota(jnp.int32, (nbb, B, bblk), 1)
        == jax.lax.broadcasted_iota(jnp.int32, (nbb, B, bblk), 2)
        + jax.lax.broadcasted_iota(jnp.int32, (nbb, B, bblk), 0) * bblk
    ).astype(jnp.float32)

    out = pl.pallas_call(
        functools.partial(_body, nbb, nsteps),
        grid=(nsteps,),
        in_specs=[
            pl.BlockSpec((dblk, E), lambda i: (i // nbb, 0)),
            pl.BlockSpec((1, dblk, B), lambda i: (i // nbb, 0, 0)),
            pl.BlockSpec((1, B, bblk), lambda i: (i % nbb, 0, 0)),
            pl.BlockSpec((1, E), lambda i: (0, 0)),
            pl.BlockSpec((1, E), lambda i: (0, 0)),
        ],
        out_specs=pl.BlockSpec(memory_space=pltpu.MemorySpace.HBM),
        out_shape=jax.ShapeDtypeStruct((B, D, E), jnp.float32),
        scratch_shapes=[
            pltpu.VMEM((_NBUF, bblk, dblk, E), jnp.float32),
            pltpu.VMEM((dblk, E), jnp.float32),
            pltpu.SemaphoreType.DMA((_NBUF,)),
        ],
    )(otu_table, clr3, sel3, w_row, b_row)
    return out


# speedup vs baseline: 1.0301x; 1.0153x over previous
import functools

import jax
import jax.numpy as jnp
from jax.experimental import pallas as pl
from jax.experimental.pallas import tpu as pltpu

_BBLK = 32
_NBUF = 2


def _body(nbb, nsteps, otu_ref, clr_ref, sel_ref, w_ref, b_ref, out_ref,
          buf_ref, tpb_ref, sems):
    i = pl.program_id(0)
    bblk = buf_ref.shape[1]
    dblk = buf_ref.shape[2]
    slot = jax.lax.rem(i, _NBUF)
    g_idx = jax.lax.rem(i, nbb)

    dst0 = out_ref.at[0, pl.ds(0, dblk), :]

    @pl.when(i >= _NBUF)
    def _wait_prev():
        for j in range(bblk):
            pltpu.make_async_copy(buf_ref.at[slot, j], dst0, sems.at[slot]).wait()

    @pl.when(g_idx == 0)
    def _fold_bias():
        tpb_ref[...] = otu_ref[...] + b_ref[...]

    blk = clr_ref[0]
    if bblk == blk.shape[1]:
        cols = blk
    else:
        cols = jnp.dot(
            blk, sel_ref[0],
            preferred_element_type=jnp.float32,
            precision=jax.lax.Precision.HIGHEST,
        )
    d_idx = i // nbb
    for j in range(bblk):
        buf_ref[slot, j] = tpb_ref[...] + cols[:, j:j + 1] * w_ref[...]
        dst = out_ref.at[g_idx * bblk + j, pl.ds(d_idx * dblk, dblk), :]
        pltpu.make_async_copy(buf_ref.at[slot, j], dst, sems.at[slot]).start()

    @pl.when(i == nsteps - 1)
    def _drain():
        for k in range(_NBUF):
            for j in range(bblk):
                pltpu.make_async_copy(buf_ref.at[k, j], dst0, sems.at[k]).wait()


def _pick_dblk(d: int) -> int:
    best = 8
    for cand in range(8, 513, 8):
        if d % cand == 0:
            best = cand
    return best


def kernel(clr, otu_table, W_val, b_val):
    B, D = clr.shape
    E = otu_table.shape[1]
    dblk = _pick_dblk(D)
    ndb = D // dblk
    bblk = _BBLK if B % _BBLK == 0 else 1
    nbb = B // bblk
    nsteps = ndb * nbb

    clr3 = clr.T.reshape(ndb, dblk, B)
    w_row = W_val[:, 0].reshape(1, E)
    b_row = b_val.reshape(1, E)
    sel3 = (
        jax.lax.broadcasted_iota(jnp.int32, (nbb, B, bblk), 1)
        == jax.lax.broadcasted_iota(jnp.int32, (nbb, B, bblk), 2)
        + jax.lax.broadcasted_iota(jnp.int32, (nbb, B, bblk), 0) * bblk
    ).astype(jnp.float32)

    out = pl.pallas_call(
        functools.partial(_body, nbb, nsteps),
        grid=(nsteps,),
        in_specs=[
            pl.BlockSpec((dblk, E), lambda i: (i // nbb, 0)),
            pl.BlockSpec((1, dblk, B), lambda i: (i // nbb, 0, 0)),
            pl.BlockSpec((1, B, bblk), lambda i: (i % nbb, 0, 0)),
            pl.BlockSpec((1, E), lambda i: (0, 0)),
            pl.BlockSpec((1, E), lambda i: (0, 0)),
        ],
        out_specs=pl.BlockSpec(memory_space=pltpu.MemorySpace.HBM),
        out_shape=jax.ShapeDtypeStruct((B, D, E), jnp.float32),
        scratch_shapes=[
            pltpu.VMEM((_NBUF, bblk, dblk, E), jnp.float32),
            pltpu.VMEM((dblk, E), jnp.float32),
            pltpu.SemaphoreType.DMA((_NBUF,)),
        ],
    )(otu_table, clr3, sel3, w_row, b_row)
    return out
